# baseline (speedup 1.0000x reference)
import functools

import jax
import jax.numpy as jnp
from jax import lax
from jax.experimental import pallas as pl
from jax.experimental.pallas import tpu as pltpu
from jax.experimental.pallas import tpu_sc as plsc

D = 64
NUM_CORES = 2
NUM_SUBCORES = 16
NW = NUM_CORES * NUM_SUBCORES
LANES = 16


def _gather_kernel(n_rows, b_rows):
    e_total = b_rows * D
    e_per_w = e_total // NW
    cols_per_w = D // NW

    @functools.partial(
        pl.kernel,
        mesh=plsc.VectorSubcoreMesh(core_axis_name="c", subcore_axis_name="s"),
        out_type=jax.ShapeDtypeStruct((e_total,), jnp.float32),
        scratch_types=[
            pltpu.VMEM((e_per_w,), jnp.int32),
            pltpu.VMEM((e_per_w,), jnp.float32),
            pltpu.SemaphoreType.DMA,
        ],
    )
    def k(tbl_hbm, idx_hbm, out_hbm, fidx_v, out_v, sem):
        wid = lax.axis_index("s") * NUM_CORES + lax.axis_index("c")
        base = wid * e_per_w
        j0 = wid * cols_per_w

        pltpu.sync_copy(idx_hbm.at[pl.ds(base, e_per_w)], fidx_v)

        def body(i, carry):
            jj = i // (b_rows // LANES)
            off = i * LANES
            v = fidx_v[pl.ds(off, LANES)]
            fidx_v[pl.ds(off, LANES)] = v + (j0 + jj) * n_rows
            return carry

        lax.fori_loop(0, e_per_w // LANES, body, 0, unroll=False)

        pltpu.async_copy(tbl_hbm.at[fidx_v], out_v, sem).wait()
        pltpu.sync_copy(out_v, out_hbm.at[pl.ds(base, e_per_w)])

    return k


def kernel(self_tensor, indices):
    n, d = self_tensor.shape
    b, d2 = indices.shape
    assert d == D and d2 == D
    tbl_cm = self_tensor.T.reshape(n * d)
    idx_cm = indices.T.reshape(b * d)
    out_cm = _gather_kernel(n, b)(tbl_cm, idx_cm)
    return out_cm.reshape(d, b).T


# speedup vs baseline: 23.9945x; 23.9945x over previous
import functools

import jax
import jax.numpy as jnp
from jax import lax
from jax.experimental import pallas as pl
from jax.experimental.pallas import tpu as pltpu
from jax.experimental.pallas import tpu_sc as plsc

D = 64
NUM_CORES = 2
NUM_SUBCORES = 16
LANES = 16
CH = 128


def _gather_kernel(n_rows, b_rows):
    cols_sc = D // NUM_CORES
    i_per_t = b_rows // NUM_SUBCORES
    n_desc = i_per_t // CH

    @functools.partial(
        pl.kernel,
        mesh=plsc.VectorSubcoreMesh(core_axis_name="c", subcore_axis_name="s"),
        out_type=jax.ShapeDtypeStruct((D, b_rows), jnp.float32),
        scratch_types=[
            pltpu.VMEM((cols_sc, i_per_t), jnp.int32),
            pltpu.VMEM((cols_sc, i_per_t), jnp.float32),
            pltpu.VMEM_SHARED((n_rows,), jnp.float32),
            pltpu.SemaphoreType.DMA,
            pltpu.SemaphoreType.DMA,
        ],
    )
    def k(tbl_hbm, idx_hbm, out_hbm, idx_v, out_v, col_sh, sem_stage, sem_g):
        c = lax.axis_index("c")
        s = lax.axis_index("s")
        j0 = c * cols_sc
        t0 = s * i_per_t

        pltpu.sync_copy(
            idx_hbm.at[pl.ds(j0, cols_sc), pl.ds(t0, i_per_t)], idx_v
        )

        def per_column(jl, carry):
            @pl.when(s == 0)
            def _stage():
                pltpu.async_copy(
                    tbl_hbm.at[j0 + jl], col_sh, sem_stage
                ).wait()

            plsc.subcore_barrier()

            copies = []
            for kd in range(n_desc):
                copies.append(
                    pltpu.async_copy(
                        col_sh.at[idx_v.at[jl, pl.ds(kd * CH, CH)]],
                        out_v.at[jl, pl.ds(kd * CH, CH)],
                        sem_g,
                    )
                )
            for cp in copies:
                cp.wait()

            plsc.subcore_barrier()
            return carry

        lax.fori_loop(0, cols_sc, per_column, 0, unroll=False)

        pltpu.sync_copy(
            out_v, out_hbm.at[pl.ds(j0, cols_sc), pl.ds(t0, i_per_t)]
        )

    return k


def kernel(self_tensor, indices):
    n, d = self_tensor.shape
    b, d2 = indices.shape
    assert d == D and d2 == D
    out_t = _gather_kernel(n, b)(self_tensor.T, indices.T)
    return out_t.T
